# baseline (speedup 1.0000x reference)
import functools

import jax
import jax.numpy as jnp
from jax import lax
from jax.experimental import pallas as pl
from jax.experimental.pallas import tpu as pltpu
from jax.experimental.pallas import tpu_sc as plsc

D_MODEL = 4096
B_TOTAL = 4 * 2048
NUM_CORES = 2
NUM_SUBCORES = 16
NUM_WORKERS = NUM_CORES * NUM_SUBCORES
B_PER_W = B_TOTAL // NUM_WORKERS
CHUNK = 8
NCHUNK = B_PER_W // CHUNK

_mesh = plsc.VectorSubcoreMesh(
    core_axis_name="c", subcore_axis_name="s",
    num_cores=NUM_CORES, num_subcores=NUM_SUBCORES)


@functools.partial(
    pl.kernel,
    out_type=jax.ShapeDtypeStruct((B_TOTAL, D_MODEL), jnp.float32),
    mesh=_mesh,
    scratch_types=[
        pltpu.VMEM((B_PER_W,), jnp.int32),
        pltpu.VMEM((CHUNK, D_MODEL), jnp.float32),
        pltpu.VMEM((CHUNK, D_MODEL), jnp.float32),
        pltpu.VMEM((CHUNK, D_MODEL), jnp.float32),
        pltpu.SemaphoreType.DMA,
        pltpu.SemaphoreType.DMA,
        pltpu.SemaphoreType.DMA,
        pltpu.SemaphoreType.DMA,
        pltpu.SemaphoreType.DMA,
        pltpu.SemaphoreType.DMA,
    ],
)
def _embed_sc(idx_hbm, table_hbm, out_hbm, idx_v,
              buf0, buf1, buf2, g0, g1, g2, w0, w1, w2):
    wid = lax.axis_index("s") * NUM_CORES + lax.axis_index("c")
    base = wid * B_PER_W
    pltpu.sync_copy(idx_hbm.at[pl.ds(base, B_PER_W)], idx_v)

    bufs = (buf0, buf1, buf2)
    gsems = (g0, g1, g2)
    wsems = (w0, w1, w2)

    def gather(c, b):
        pltpu.async_copy(
            table_hbm.at[idx_v.at[pl.ds(c * CHUNK, CHUNK)]], bufs[b], gsems[b])

    def wait_gather(b):
        pltpu.make_async_copy(
            table_hbm.at[pl.ds(0, CHUNK)], bufs[b], gsems[b]).wait()

    def write(c, b):
        pltpu.async_copy(bufs[b], out_hbm.at[pl.ds(base + c * CHUNK, CHUNK)],
                         wsems[b])

    def wait_write(b):
        pltpu.make_async_copy(
            bufs[b], out_hbm.at[pl.ds(0, CHUNK)], wsems[b]).wait()

    gather(0, 1)
    gather(1, 2)
    gather(2, 0)
    wait_gather(1)
    write(0, 1)
    wait_gather(2)
    write(1, 2)

    def round_(r, _):
        for j in range(3):
            c = 2 + 3 * r + j
            nb = (j + 1) % 3

            @pl.when(c + 1 < NCHUNK)
            def _():
                wait_write(nb)
                gather(c + 1, nb)

            wait_gather(j)
            write(c, j)
        return _

    lax.fori_loop(0, (NCHUNK - 2) // 3, round_, None)

    wait_write(0)
    wait_write(1)
    wait_write(2)


def kernel(input_ids, table):
    ids_flat = input_ids.reshape(-1)
    out = _embed_sc(ids_flat, table)
    return out.reshape(input_ids.shape + (table.shape[1],))


# speedup vs baseline: 1.0052x; 1.0052x over previous
import functools

import jax
import jax.numpy as jnp
from jax import lax
from jax.experimental import pallas as pl
from jax.experimental.pallas import tpu as pltpu
from jax.experimental.pallas import tpu_sc as plsc

D_MODEL = 4096
B_TOTAL = 4 * 2048
NUM_CORES = 2
NUM_SUBCORES = 16
NUM_WORKERS = NUM_CORES * NUM_SUBCORES
B_PER_W = B_TOTAL // NUM_WORKERS
CHUNK = 4
NCHUNK = B_PER_W // CHUNK
NBUF = 6
AHEAD = 4

_mesh = plsc.VectorSubcoreMesh(
    core_axis_name="c", subcore_axis_name="s",
    num_cores=NUM_CORES, num_subcores=NUM_SUBCORES)


@functools.partial(
    pl.kernel,
    out_type=jax.ShapeDtypeStruct((B_TOTAL, D_MODEL), jnp.float32),
    mesh=_mesh,
    scratch_types=[
        pltpu.VMEM((NCHUNK, CHUNK), jnp.int32),
        pltpu.VMEM((NBUF, CHUNK, D_MODEL), jnp.float32),
        [pltpu.SemaphoreType.DMA] * NBUF,
        [pltpu.SemaphoreType.DMA] * NBUF,
    ],
)
def _embed_sc(idx_hbm, table_hbm, out_hbm, idx_v, bufs, gsems, wsems):
    wid = lax.axis_index("s") * NUM_CORES + lax.axis_index("c")
    base = wid * B_PER_W
    pltpu.sync_copy(idx_hbm.at[pl.ds(wid * NCHUNK, NCHUNK)], idx_v)

    def gather(c, b):
        pltpu.async_copy(table_hbm.at[idx_v.at[c]], bufs.at[b], gsems[b])

    def wait_gather(b):
        pltpu.make_async_copy(
            table_hbm.at[pl.ds(0, CHUNK)], bufs.at[b], gsems[b]).wait()

    def write(c, b):
        pltpu.async_copy(bufs.at[b], out_hbm.at[pl.ds(base + c * CHUNK, CHUNK)],
                         wsems[b])

    def wait_write(b):
        pltpu.make_async_copy(
            bufs.at[b], out_hbm.at[pl.ds(0, CHUNK)], wsems[b]).wait()

    for c0 in range(AHEAD):
        gather(c0, c0)

    def round_(r, _):
        for j in range(NBUF):
            c = NBUF * r + j
            bn = (j + AHEAD) % NBUF

            @pl.when(c >= 2)
            def _():
                wait_write(bn)

            gather(c + AHEAD, bn)
            wait_gather(j)
            write(c, j)
        return _

    lax.fori_loop(0, (NCHUNK - AHEAD) // NBUF, round_, None)

    for cc in range(NCHUNK - AHEAD, NCHUNK):
        b = cc % NBUF
        wait_gather(b)
        write(cc, b)

    for cc in range(NCHUNK - NBUF, NCHUNK):
        wait_write(cc % NBUF)


def kernel(input_ids, table):
    ids_2d = input_ids.reshape(-1, CHUNK)
    out = _embed_sc(ids_2d, table)
    return out.reshape(input_ids.shape + (table.shape[1],))


# speedup vs baseline: 1.0249x; 1.0196x over previous
import functools

import jax
import jax.numpy as jnp
from jax import lax
from jax.experimental import pallas as pl
from jax.experimental.pallas import tpu as pltpu
from jax.experimental.pallas import tpu_sc as plsc

D_MODEL = 4096
B_TOTAL = 4 * 2048
NUM_CORES = 2
NUM_SUBCORES = 16
NUM_WORKERS = NUM_CORES * NUM_SUBCORES
B_PER_W = B_TOTAL // NUM_WORKERS
CHUNK = 4
NCHUNK = B_PER_W // CHUNK
NBUF = 4
NSLOT = 2
AHEAD = 2

_mesh = plsc.VectorSubcoreMesh(
    core_axis_name="c", subcore_axis_name="s",
    num_cores=NUM_CORES, num_subcores=NUM_SUBCORES)


@functools.partial(
    pl.kernel,
    out_type=jax.ShapeDtypeStruct((B_TOTAL, D_MODEL), jnp.float32),
    mesh=_mesh,
    scratch_types=[
        pltpu.VMEM((NCHUNK, CHUNK), jnp.int32),
        pltpu.VMEM((NBUF, CHUNK, D_MODEL), jnp.float32),
        pltpu.VMEM_SHARED((NUM_SUBCORES, NSLOT, CHUNK, D_MODEL), jnp.float32),
        [pltpu.SemaphoreType.DMA] * NBUF,
        [pltpu.SemaphoreType.DMA] * NBUF,
        [pltpu.SemaphoreType.DMA] * NSLOT,
    ],
)
def _embed_sc(idx_hbm, table_hbm, out_hbm, idx_v, bufs, sh,
              gsems, xsems, wsems):
    sid = lax.axis_index("s")
    wid = sid * NUM_CORES + lax.axis_index("c")
    base = wid * B_PER_W
    pltpu.sync_copy(idx_hbm.at[pl.ds(wid * NCHUNK, NCHUNK)], idx_v)

    def gather(c, b):
        pltpu.async_copy(table_hbm.at[idx_v.at[c]], bufs.at[b], gsems[b])

    def wait_gather(b):
        pltpu.make_async_copy(
            table_hbm.at[pl.ds(0, CHUNK)], bufs.at[b], gsems[b]).wait()

    def xbar(b, slot):
        pltpu.async_copy(bufs.at[b], sh.at[sid, slot], xsems[b])

    def wait_x(b):
        pltpu.make_async_copy(
            table_hbm.at[pl.ds(0, CHUNK)], bufs.at[b], xsems[b]).wait()

    def write(c, slot):
        pltpu.async_copy(sh.at[sid, slot],
                         out_hbm.at[pl.ds(base + c * CHUNK, CHUNK)],
                         wsems[slot])

    def wait_write(slot):
        pltpu.make_async_copy(
            sh.at[sid, slot], out_hbm.at[pl.ds(0, CHUNK)], wsems[slot]).wait()

    for c0 in range(AHEAD):
        gather(c0, c0)

    def round_(r, _):
        for j in range(NBUF):
            c = NBUF * r + j
            bn = (j + AHEAD) % NBUF
            sl = j % NSLOT

            @pl.when(c + AHEAD < NCHUNK)
            def _():
                gather(c + AHEAD, bn)

            wait_gather(j)

            @pl.when(c >= AHEAD)
            def _():
                wait_write(sl)

            xbar(j, sl)
            wait_x(j)
            write(c, sl)
        return _

    lax.fori_loop(0, NCHUNK // NBUF, round_, None)

    wait_write(0)
    wait_write(1)


def kernel(input_ids, table):
    ids_2d = input_ids.reshape(-1, CHUNK)
    out = _embed_sc(ids_2d, table)
    return out.reshape(input_ids.shape + (table.shape[1],))
